# baseline (speedup 1.0000x reference)
import functools
import jax
import jax.numpy as jnp
from jax.experimental import pallas as pl
from jax.experimental.pallas import tpu as pltpu

_LANE = 128


def _moe_body(n_routed, n_inter_chunks, x_ref, w1_ref, w2_ref, rwt_ref,
              out_ref, c_ref):
    e = pl.program_id(1)
    ib = pl.program_id(2)
    xb = x_ref[...]

    @pl.when(ib == 0)
    def _compute_combined_weight():
        logits = jnp.dot(xb, rwt_ref[...], preferred_element_type=jnp.float32)
        lane = jax.lax.broadcasted_iota(jnp.int32, logits.shape, 1)
        neg = jnp.float32(-1e30)
        l = jnp.where(lane < n_routed, logits, neg)
        m0 = jnp.max(l, axis=1, keepdims=True)
        i0 = jnp.min(jnp.where(l == m0, lane, 9999), axis=1, keepdims=True)
        l2 = jnp.where(lane == i0, neg, l)
        m1 = jnp.max(l2, axis=1, keepdims=True)
        i1 = jnp.min(jnp.where(l2 == m1, lane, 9999), axis=1, keepdims=True)
        c0 = jax.nn.sigmoid(m0 - m1)
        c = jnp.where(i0 == e, c0, 0.0) + jnp.where(i1 == e, 1.0 - c0, 0.0)
        c = jnp.where(e >= n_routed, 1.0, c)
        c_ref[...] = jnp.broadcast_to(c, c_ref.shape)

    w1c = w1_ref[0]
    w2c = w2_ref[0]
    h = jax.lax.dot_general(xb, w1c, (((1,), (1,)), ((), ())),
                            preferred_element_type=jnp.float32)
    h = h * jax.nn.sigmoid(h)
    h = h.astype(w2c.dtype)
    y = jax.lax.dot_general(h, w2c, (((1,), (1,)), ((), ())),
                            preferred_element_type=jnp.float32)
    y = y * c_ref[:, 0:1]

    @pl.when((e == 0) & (ib == 0))
    def _init():
        out_ref[...] = y

    @pl.when((e > 0) | (ib > 0))
    def _acc():
        out_ref[...] += y


def kernel(hidden_states, shared_w1, shared_w2, routed_w1, routed_w2,
           router_w):
    bsz, seq, hdim = hidden_states.shape
    T = bsz * seq
    n_routed, inter, _ = routed_w1.shape
    n_shared = shared_w1.shape[0]
    n_exp = n_routed + n_shared

    x = hidden_states.reshape(T, hdim).astype(jnp.bfloat16)
    w1 = jnp.concatenate([routed_w1, shared_w1], axis=0).astype(jnp.bfloat16)
    w2 = jnp.concatenate([routed_w2, shared_w2], axis=0).astype(jnp.bfloat16)
    rwt = jnp.zeros((hdim, _LANE), jnp.float32).at[:, :n_routed].set(
        router_w.T).astype(jnp.bfloat16)

    TB = 1024 if T % 1024 == 0 else T
    IB = 1024 if inter % 1024 == 0 else inter
    n_tb = T // TB
    n_ib = inter // IB

    body = functools.partial(_moe_body, n_routed, n_ib)
    out = pl.pallas_call(
        body,
        grid=(n_tb, n_exp, n_ib),
        in_specs=[
            pl.BlockSpec((TB, hdim), lambda t, e, ib: (t, 0)),
            pl.BlockSpec((1, IB, hdim), lambda t, e, ib: (e, ib, 0)),
            pl.BlockSpec((1, hdim, IB), lambda t, e, ib: (e, 0, ib)),
            pl.BlockSpec((hdim, _LANE), lambda t, e, ib: (0, 0)),
        ],
        out_specs=pl.BlockSpec((TB, hdim), lambda t, e, ib: (t, 0)),
        out_shape=jax.ShapeDtypeStruct((T, hdim), jnp.float32),
        scratch_shapes=[pltpu.VMEM((TB, _LANE), jnp.float32)],
        compiler_params=pltpu.CompilerParams(
            dimension_semantics=("parallel", "arbitrary", "arbitrary")),
    )(x, w1, w2, rwt)
    return out.reshape(bsz, seq, hdim)


# speedup vs baseline: 1.5669x; 1.5669x over previous
import functools
import jax
import jax.numpy as jnp
from jax import lax
from jax.experimental import pallas as pl
from jax.experimental.pallas import tpu as pltpu
from jax.experimental.pallas import tpu_sc as plsc

NC, NS, L = 2, 16, 16
NT = NC * NS
TOPK = 2
TB = 512
IB = 2048
SKIP = 9


def _f32(x):
    return x.astype(jnp.float32)


def _iota():
    return lax.iota(jnp.int32, L)


def _perm16(v, idx):
    return v.at[idx].get(mode="promise_in_bounds")


def _csum16(v):
    io = _iota()
    zeros = jnp.zeros((L,), jnp.int32)
    r = v
    sh = 1
    while sh < L:
        shv = jnp.full((L,), sh, jnp.int32)
        sv = _perm16(r, jnp.maximum(io - shv, zeros))
        r = r + jnp.where(io >= shv, sv, zeros)
        sh *= 2
    return r


def _bcast16(v, i):
    return _perm16(v, jnp.full((L,), i, jnp.int32))


def _k1_body(rw_ref, x_ref, lt_ref):
    lt_ref[...] = lax.dot_general(
        rw_ref[...], x_ref[...], (((1,), (1,)), ((), ())),
        preferred_element_type=jnp.float32)


def _k1_router_logits(x, router_w, T, H, E):
    TBK = 2048
    return pl.pallas_call(
        _k1_body,
        grid=(T // TBK,),
        in_specs=[
            pl.BlockSpec((E, H), lambda t: (0, 0)),
            pl.BlockSpec((TBK, H), lambda t: (t, 0)),
        ],
        out_specs=pl.BlockSpec((E, TBK), lambda t: (0, t)),
        out_shape=jax.ShapeDtypeStruct((E, T), jnp.float32),
    )(router_w, x)


def _k2a_body(E, TPT, lt_hbm, idx01, w01, cnt, lbuf, i0b, i1b, w0b, w1b,
              cbuf):
    wid = lax.axis_index("s") * NC + lax.axis_index("c")
    t0 = wid * TPT
    for e in range(E):
        pltpu.sync_copy(lt_hbm.at[e, pl.ds(t0, TPT)], lbuf.at[e])

    lane = lax.iota(jnp.int32, L)
    zeros = jnp.zeros((L,), jnp.int32)

    ones = jnp.ones((L,), jnp.float32)
    neg = jnp.full((L,), -3e38, jnp.float32)
    evecs = [jnp.full((L,), e, jnp.int32) for e in range(E)]

    def group(g, accs):
        off = g * L
        ls = [lbuf[e, pl.ds(off, L)] for e in range(E)]
        m0 = ls[0]
        for e in range(1, E):
            m0 = jnp.maximum(m0, ls[e])
        i0 = jnp.full((L,), 99, jnp.int32)
        for e in range(E - 1, -1, -1):
            i0 = jnp.where(ls[e] == m0, evecs[e], i0)
        m1 = neg
        for e in range(E):
            le = jnp.where(i0 == evecs[e], neg, ls[e])
            m1 = jnp.maximum(m1, le)
        i1 = jnp.full((L,), 99, jnp.int32)
        for e in range(E - 1, -1, -1):
            i1 = jnp.where((i0 != evecs[e]) & (ls[e] == m1), evecs[e], i1)
        c0 = ones / (ones + jnp.exp(m1 - m0))
        i0b[pl.ds(off, L)] = i0
        i1b[pl.ds(off, L)] = i1
        w0b[pl.ds(off, L)] = c0
        w1b[pl.ds(off, L)] = ones - c0
        ivec = jnp.ones((L,), jnp.int32)
        zvec = jnp.zeros((L,), jnp.int32)
        new = [accs[e]
               + jnp.where(i0 == evecs[e], ivec, zvec)
               + jnp.where(i1 == evecs[e], ivec, zvec)
               for e in range(E)]
        return new

    accs = [jnp.zeros((L,), jnp.int32) for _ in range(E)]
    accs = lax.fori_loop(0, TPT // L, group, accs)
    lane = _iota()
    zeros = jnp.zeros((L,), jnp.int32)
    cvec = zeros
    for e in range(E):
        tot = _bcast16(_csum16(accs[e]), L - 1)
        cvec = cvec + jnp.where(lane == evecs[e], tot, zeros)
    cbuf[...] = cvec
    pltpu.sync_copy(i0b, idx01.at[0, pl.ds(t0, TPT)])
    pltpu.sync_copy(i1b, idx01.at[1, pl.ds(t0, TPT)])
    pltpu.sync_copy(w0b, w01.at[0, pl.ds(t0, TPT)])
    pltpu.sync_copy(w1b, w01.at[1, pl.ds(t0, TPT)])
    pltpu.sync_copy(cbuf, cnt.at[wid])


def _k2a_route(logitsT, T, E):
    TPT = T // NT
    mesh = plsc.VectorSubcoreMesh(core_axis_name="c", subcore_axis_name="s")
    body = functools.partial(_k2a_body, E, TPT)
    fn = pl.kernel(
        body,
        out_type=[
            jax.ShapeDtypeStruct((TOPK, T), jnp.int32),
            jax.ShapeDtypeStruct((TOPK, T), jnp.float32),
            jax.ShapeDtypeStruct((NT, L), jnp.int32),
        ],
        mesh=mesh,
        compiler_params=pltpu.CompilerParams(
            needs_layout_passes=False),
        scratch_types=[
            pltpu.VMEM((E, TPT), jnp.float32),
            pltpu.VMEM((TPT,), jnp.int32),
            pltpu.VMEM((TPT,), jnp.int32),
            pltpu.VMEM((TPT,), jnp.float32),
            pltpu.VMEM((TPT,), jnp.float32),
            pltpu.VMEM((L,), jnp.int32),
        ],
    )
    return fn(logitsT)


def _k2b_body(E, TPT, T, P_rt, cnt, idx01h, w01h, comb, pos, gmap,
              cntall, ibuf, wbuf, posb, spos2, scomb, gmapb, sem):
    wid = lax.axis_index("s") * NC + lax.axis_index("c")
    t0 = wid * TPT
    pltpu.sync_copy(cnt, cntall)
    for k in range(TOPK):
        pltpu.sync_copy(idx01h.at[k, pl.ds(t0, TPT)], ibuf.at[k])
        pltpu.sync_copy(w01h.at[k, pl.ds(t0, TPT)], wbuf.at[k])

    lane = _iota()
    zeros = jnp.zeros((L,), jnp.int32)
    ivec = jnp.ones((L,), jnp.int32)
    evecs = [jnp.full((L,), e, jnp.int32) for e in range(E)]

    totals = lax.fori_loop(
        0, NT, lambda t, a: a + cntall[t], zeros)
    mybase = lax.fori_loop(
        0, wid, lambda t, a: a + cntall[t], zeros)
    padded = ((totals + jnp.full((L,), TB - 1, jnp.int32))
              // jnp.full((L,), TB, jnp.int32)) * jnp.full((L,), TB,
                                                          jnp.int32)
    offs = _csum16(padded) - padded
    base0 = offs + mybase

    ngrp = TPT // L

    def assign(k, basevec):
        def body(g, basevec):
            off = g * L
            idv = ibuf[k, pl.ds(off, L)]
            wv = wbuf[k, pl.ds(off, L)]
            posv = zeros
            for e in range(E):
                mask = idv == evecs[e]
                cums = _csum16(jnp.where(mask, ivec, zeros))
                pc = _bcast16(cums, L - 1)
                bvec = _bcast16(basevec, e)
                posv = posv + jnp.where(mask, bvec + cums - ivec, zeros)
                basevec = basevec + jnp.where(lane == evecs[e], pc, zeros)
            posb[k, pl.ds(off, L)] = posv + jnp.full((L,), T, jnp.int32)
            j = k * TPT + off
            spos2[j // 128, pl.ds(j % 128, L)] = posv
            row = jnp.full((L,), j, jnp.int32) + lane
            tok = jnp.full((L,), t0, jnp.int32) + jnp.full(
                (L,), off, jnp.int32) + lane
            plsc.store_scatter(scomb, [row, zeros], tok)
            plsc.store_scatter(scomb, [row, ivec],
                               plsc.bitcast(wv, jnp.int32))
            return basevec
        return lax.fori_loop(0, ngrp, body, basevec)

    basevec = base0
    for k in range(TOPK):
        basevec = assign(k, basevec)

    NCH = (TOPK * TPT) // 128
    for j in range(NCH):
        pltpu.async_copy(scomb.at[pl.ds(j * 128, 128)],
                         comb.at[spos2.at[j]], sem).wait()
    for k in range(TOPK):
        pltpu.sync_copy(posb.at[k], pos.at[k, pl.ds(t0, TPT)])

    @pl.when(wid == 0)
    def _gmap():
        n_sh = T // TB
        for v in range(64 // L):
            iv = lane + jnp.full((L,), v * L, jnp.int32)
            s_rt = (iv - jnp.full((L,), n_sh, jnp.int32)) * jnp.full(
                (L,), TB, jnp.int32)
            g = jnp.full((L,), SKIP, jnp.int32)
            for e in range(E):
                lo = _bcast16(offs, e)
                hi = lo + _bcast16(padded, e)
                g = jnp.where((s_rt >= lo) & (s_rt < hi), evecs[e], g)
            g = jnp.where(iv < jnp.full((L,), n_sh, jnp.int32),
                          jnp.full((L,), E, jnp.int32), g)
            gmapb[pl.ds(v * L, L)] = g
        pltpu.sync_copy(gmapb, gmap)


def _k2b_dispatch(cnt, idx01, w01, T, E, P_rt):
    TPT = T // NT
    mesh = plsc.VectorSubcoreMesh(core_axis_name="c", subcore_axis_name="s")
    body = functools.partial(_k2b_body, E, TPT, T, P_rt)
    fn = pl.kernel(
        body,
        out_type=[
            jax.ShapeDtypeStruct((P_rt, 128), jnp.int32),
            jax.ShapeDtypeStruct((TOPK, T), jnp.int32),
            jax.ShapeDtypeStruct((64,), jnp.int32),
        ],
        mesh=mesh,
        compiler_params=pltpu.CompilerParams(
            needs_layout_passes=False),
        scratch_types=[
            pltpu.VMEM((NT, L), jnp.int32),
            pltpu.VMEM((TOPK, TPT), jnp.int32),
            pltpu.VMEM((TOPK, TPT), jnp.float32),
            pltpu.VMEM((TOPK, TPT), jnp.int32),
            pltpu.VMEM(((TOPK * TPT) // 128, 128), jnp.int32),
            pltpu.VMEM((TOPK * TPT, 128), jnp.int32),
            pltpu.VMEM((64,), jnp.int32),
            pltpu.SemaphoreType.DMA,
        ],
    )
    return fn(cnt, idx01, w01)


def _k3_body(T, H, P_rt, RPT, comb, xh, xs, combc, idxc, rbuf, sem):
    wid = lax.axis_index("s") * NC + lax.axis_index("c")
    TPT = T // NT
    t0 = wid * TPT
    CH = 64
    for c in range(TPT // CH):
        pltpu.sync_copy(xh.at[pl.ds(t0 + c * CH, CH)], rbuf)
        pltpu.sync_copy(rbuf, xs.at[pl.ds(t0 + c * CH, CH)])
    r0 = wid * RPT
    zeros = jnp.zeros((L,), jnp.int32)
    tmax = jnp.full((L,), T - 1, jnp.int32)

    def gat(c, _):
        pltpu.sync_copy(comb.at[pl.ds(r0 + c * CH, CH)], combc)
        for g in range(CH // L):
            rvec = jnp.full((L,), g * L, jnp.int32) + _iota()
            tok = plsc.load_gather(combc, [rvec, zeros])
            idxc[pl.ds(g * L, L)] = jnp.minimum(jnp.maximum(tok, zeros),
                                                tmax)
        pltpu.async_copy(xh.at[idxc], rbuf, sem).wait()
        pltpu.sync_copy(rbuf, xs.at[pl.ds(T + r0 + c * CH, CH)])
        return 0

    lax.fori_loop(0, RPT // CH, gat, 0)


def _k3_gather(comb, x, T, H, P_rt):
    RPT = P_rt // NT
    mesh = plsc.VectorSubcoreMesh(core_axis_name="c", subcore_axis_name="s")
    body = functools.partial(_k3_body, T, H, P_rt, RPT)
    fn = pl.kernel(
        body,
        out_type=jax.ShapeDtypeStruct((T + P_rt, H), jnp.float32),
        mesh=mesh,
        compiler_params=pltpu.CompilerParams(
            needs_layout_passes=False),
        scratch_types=[
            pltpu.VMEM((64, 128), jnp.int32),
            pltpu.VMEM((64,), jnp.int32),
            pltpu.VMEM((64, H), jnp.float32),
            pltpu.SemaphoreType.DMA,
        ],
    )
    return fn(comb, x)


def _k4_body(E, n_sh, gm_ref, xs_ref, w1_ref, w2_ref, cw_ref, ys_ref):
    t = pl.program_id(0)
    ib = pl.program_id(1)
    g = gm_ref[t]

    @pl.when(g != SKIP)
    def _compute():
        xb = xs_ref[...]
        w1c = w1_ref[0]
        w2c = w2_ref[0]
        h = lax.dot_general(xb, w1c, (((1,), (1,)), ((), ())),
                            preferred_element_type=jnp.float32)
        h = h * (1.0 / (1.0 + jnp.exp(-h)))
        y = lax.dot_general(h, w2c, (((1,), (1,)), ((), ())),
                            preferred_element_type=jnp.float32)
        c = jnp.where(
            t < n_sh, 1.0,
            lax.bitcast_convert_type(cw_ref[:, 1:2], jnp.float32))
        y = y * c

        @pl.when(ib == 0)
        def _init():
            ys_ref[...] = y

        @pl.when(ib > 0)
        def _acc():
            ys_ref[...] += y


def _k4_ffn(gmap, xs, w1, w2, comb, T, H, I, P_rt):
    n_sh = T // TB
    n_tiles = (T + P_rt) // TB
    n_ib = I // IB
    body = functools.partial(_k4_body, w1.shape[0], n_sh)
    grid_spec = pltpu.PrefetchScalarGridSpec(
        num_scalar_prefetch=1,
        grid=(n_tiles, n_ib),
        in_specs=[
            pl.BlockSpec((TB, H), lambda t, ib, gm: (t, 0)),
            pl.BlockSpec((1, IB, H),
                         lambda t, ib, gm: (jnp.minimum(gm[t], 8), ib, 0)),
            pl.BlockSpec((1, H, IB),
                         lambda t, ib, gm: (jnp.minimum(gm[t], 8), 0, ib)),
            pl.BlockSpec((TB, 128),
                         lambda t, ib, gm: (jnp.maximum(t - n_sh, 0), 0)),
        ],
        out_specs=pl.BlockSpec((TB, H), lambda t, ib, gm: (t, 0)),
    )
    return pl.pallas_call(
        body,
        grid_spec=grid_spec,
        out_shape=jax.ShapeDtypeStruct((T + P_rt, H), jnp.float32),
        compiler_params=pltpu.CompilerParams(
            dimension_semantics=("arbitrary", "arbitrary")),
    )(gmap, xs, w1, w2, comb)


def _k5_body(T, H, P, ys, pos, out, praw, p0c, p1c, bbuf, g0, g1, sem0,
             sem1):
    wid = lax.axis_index("s") * NC + lax.axis_index("c")
    TPT = T // NT
    t0 = wid * TPT
    CH = 32
    nch = TPT // CH

    pltpu.sync_copy(pos.at[0, pl.ds(t0, TPT)], praw.at[0])
    pltpu.sync_copy(pos.at[1, pl.ds(t0, TPT)], praw.at[1])

    def ldpos(c, _):
        for g in range(CH // L):
            o = c * CH + g * L
            p0 = praw[0, pl.ds(o, L)]
            p1 = praw[1, pl.ds(o, L)]
            p0c[c, pl.ds(g * L, L)] = jnp.clip(p0, 0, P - 1)
            p1c[c, pl.ds(g * L, L)] = jnp.clip(p1, 0, P - 1)
        return 0

    lax.fori_loop(0, nch, ldpos, 0)

    def chunk(c, _):
        cp0 = pltpu.async_copy(ys.at[p0c.at[c]], g0, sem0)
        cp1 = pltpu.async_copy(ys.at[p1c.at[c]], g1, sem1)
        pltpu.sync_copy(ys.at[pl.ds(t0 + c * CH, CH)], bbuf)
        cp0.wait()
        cp1.wait()

        def add(i, _):
            r = i // (H // L)
            o = (i % (H // L)) * L
            bbuf[r, pl.ds(o, L)] = (bbuf[r, pl.ds(o, L)]
                                    + g0[r, pl.ds(o, L)]
                                    + g1[r, pl.ds(o, L)])
            return 0

        lax.fori_loop(0, CH * (H // L), add, 0)
        pltpu.sync_copy(bbuf, out.at[pl.ds(t0 + c * CH, CH)])
        return 0

    lax.fori_loop(0, nch, chunk, 0)


def _k5_combine(ys, pos, T, H, P):
    TPT = T // NT
    CH = 32
    mesh = plsc.VectorSubcoreMesh(core_axis_name="c", subcore_axis_name="s")
    body = functools.partial(_k5_body, T, H, P)
    fn = pl.kernel(
        body,
        out_type=jax.ShapeDtypeStruct((T, H), jnp.float32),
        mesh=mesh,
        compiler_params=pltpu.CompilerParams(
            needs_layout_passes=False),
        scratch_types=[
            pltpu.VMEM((TOPK, TPT), jnp.int32),
            pltpu.VMEM((TPT // CH, CH), jnp.int32),
            pltpu.VMEM((TPT // CH, CH), jnp.int32),
            pltpu.VMEM((CH, H), jnp.float32),
            pltpu.VMEM((CH, H), jnp.float32),
            pltpu.VMEM((CH, H), jnp.float32),
            pltpu.SemaphoreType.DMA,
            pltpu.SemaphoreType.DMA,
        ],
    )
    return fn(ys, pos)


def kernel(hidden_states, shared_w1, shared_w2, routed_w1, routed_w2,
           router_w):
    bsz, seq, H = hidden_states.shape
    T = bsz * seq
    E, I, _ = routed_w1.shape
    P_rt = TOPK * T + E * TB

    x = hidden_states.reshape(T, H)
    w1 = jnp.concatenate([routed_w1, shared_w1], axis=0)
    w2 = jnp.concatenate([routed_w2, shared_w2], axis=0)

    logitsT = _k1_router_logits(x, router_w, T, H, E)
    idx01, w01, cnt = _k2a_route(logitsT, T, E)
    comb, pos, gmap = _k2b_dispatch(cnt, idx01, w01, T, E, P_rt)
    xs = _k3_gather(comb, x, T, H, P_rt)
    comb, pos, gmap = _k2b_dispatch(cnt, idx01, w01, T, E, P_rt)
    xs = _k3_gather(comb, x, T, H, P_rt)
    ys = _k4_ffn(gmap, xs, w1, w2, comb, T, H, I, P_rt)
    out = _k5_combine(ys, pos, T, H, T + P_rt)
    return out.reshape(bsz, seq, H)


# speedup vs baseline: 1.6457x; 1.0503x over previous
import functools
import jax
import jax.numpy as jnp
from jax import lax
from jax.experimental import pallas as pl
from jax.experimental.pallas import tpu as pltpu
from jax.experimental.pallas import tpu_sc as plsc

NC, NS, L = 2, 16, 16
NT = NC * NS
TOPK = 2
TB = 512
IB = 2048
SKIP = 9


def _f32(x):
    return x.astype(jnp.float32)


def _iota():
    return lax.iota(jnp.int32, L)


def _perm16(v, idx):
    return v.at[idx].get(mode="promise_in_bounds")


def _csum16(v):
    io = _iota()
    zeros = jnp.zeros((L,), jnp.int32)
    r = v
    sh = 1
    while sh < L:
        shv = jnp.full((L,), sh, jnp.int32)
        sv = _perm16(r, jnp.maximum(io - shv, zeros))
        r = r + jnp.where(io >= shv, sv, zeros)
        sh *= 2
    return r


def _bcast16(v, i):
    return _perm16(v, jnp.full((L,), i, jnp.int32))


def _k1_body(rw_ref, x_ref, lt_ref):
    lt_ref[...] = lax.dot_general(
        rw_ref[...], x_ref[...], (((1,), (1,)), ((), ())),
        preferred_element_type=jnp.float32)


def _k1_router_logits(x, router_w, T, H, E):
    TBK = 2048
    return pl.pallas_call(
        _k1_body,
        grid=(T // TBK,),
        in_specs=[
            pl.BlockSpec((E, H), lambda t: (0, 0)),
            pl.BlockSpec((TBK, H), lambda t: (t, 0)),
        ],
        out_specs=pl.BlockSpec((E, TBK), lambda t: (0, t)),
        out_shape=jax.ShapeDtypeStruct((E, T), jnp.float32),
    )(router_w, x)


def _k2a_body(E, TPT, lt_hbm, idx01, w01, cnt, lbuf, i0b, i1b, w0b, w1b,
              cbuf):
    wid = lax.axis_index("s") * NC + lax.axis_index("c")
    t0 = wid * TPT
    for e in range(E):
        pltpu.sync_copy(lt_hbm.at[e, pl.ds(t0, TPT)], lbuf.at[e])

    lane = lax.iota(jnp.int32, L)
    zeros = jnp.zeros((L,), jnp.int32)

    ones = jnp.ones((L,), jnp.float32)
    neg = jnp.full((L,), -3e38, jnp.float32)
    evecs = [jnp.full((L,), e, jnp.int32) for e in range(E)]

    def group(g, accs):
        off = g * L
        ls = [lbuf[e, pl.ds(off, L)] for e in range(E)]
        m0 = ls[0]
        for e in range(1, E):
            m0 = jnp.maximum(m0, ls[e])
        i0 = jnp.full((L,), 99, jnp.int32)
        for e in range(E - 1, -1, -1):
            i0 = jnp.where(ls[e] == m0, evecs[e], i0)
        m1 = neg
        for e in range(E):
            le = jnp.where(i0 == evecs[e], neg, ls[e])
            m1 = jnp.maximum(m1, le)
        i1 = jnp.full((L,), 99, jnp.int32)
        for e in range(E - 1, -1, -1):
            i1 = jnp.where((i0 != evecs[e]) & (ls[e] == m1), evecs[e], i1)
        c0 = ones / (ones + jnp.exp(m1 - m0))
        i0b[pl.ds(off, L)] = i0
        i1b[pl.ds(off, L)] = i1
        w0b[pl.ds(off, L)] = c0
        w1b[pl.ds(off, L)] = ones - c0
        ivec = jnp.ones((L,), jnp.int32)
        zvec = jnp.zeros((L,), jnp.int32)
        new = [accs[e]
               + jnp.where(i0 == evecs[e], ivec, zvec)
               + jnp.where(i1 == evecs[e], ivec, zvec)
               for e in range(E)]
        return new

    accs = [jnp.zeros((L,), jnp.int32) for _ in range(E)]
    accs = lax.fori_loop(0, TPT // L, group, accs)
    lane = _iota()
    zeros = jnp.zeros((L,), jnp.int32)
    cvec = zeros
    for e in range(E):
        tot = _bcast16(_csum16(accs[e]), L - 1)
        cvec = cvec + jnp.where(lane == evecs[e], tot, zeros)
    cbuf[...] = cvec
    pltpu.sync_copy(i0b, idx01.at[0, pl.ds(t0, TPT)])
    pltpu.sync_copy(i1b, idx01.at[1, pl.ds(t0, TPT)])
    pltpu.sync_copy(w0b, w01.at[0, pl.ds(t0, TPT)])
    pltpu.sync_copy(w1b, w01.at[1, pl.ds(t0, TPT)])
    pltpu.sync_copy(cbuf, cnt.at[wid])


def _k2a_route(logitsT, T, E):
    TPT = T // NT
    mesh = plsc.VectorSubcoreMesh(core_axis_name="c", subcore_axis_name="s")
    body = functools.partial(_k2a_body, E, TPT)
    fn = pl.kernel(
        body,
        out_type=[
            jax.ShapeDtypeStruct((TOPK, T), jnp.int32),
            jax.ShapeDtypeStruct((TOPK, T), jnp.float32),
            jax.ShapeDtypeStruct((NT, L), jnp.int32),
        ],
        mesh=mesh,
        compiler_params=pltpu.CompilerParams(
            needs_layout_passes=False),
        scratch_types=[
            pltpu.VMEM((E, TPT), jnp.float32),
            pltpu.VMEM((TPT,), jnp.int32),
            pltpu.VMEM((TPT,), jnp.int32),
            pltpu.VMEM((TPT,), jnp.float32),
            pltpu.VMEM((TPT,), jnp.float32),
            pltpu.VMEM((L,), jnp.int32),
        ],
    )
    return fn(logitsT)


def _k2b_body(E, TPT, T, P_rt, cnt, idx01h, w01h, comb, pos, gmap,
              cntall, ibuf, wbuf, posb, spos2, scomb, gmapb, sem):
    wid = lax.axis_index("s") * NC + lax.axis_index("c")
    t0 = wid * TPT
    pltpu.sync_copy(cnt, cntall)
    for k in range(TOPK):
        pltpu.sync_copy(idx01h.at[k, pl.ds(t0, TPT)], ibuf.at[k])
        pltpu.sync_copy(w01h.at[k, pl.ds(t0, TPT)], wbuf.at[k])

    lane = _iota()
    zeros = jnp.zeros((L,), jnp.int32)
    ivec = jnp.ones((L,), jnp.int32)
    evecs = [jnp.full((L,), e, jnp.int32) for e in range(E)]

    totals = lax.fori_loop(
        0, NT, lambda t, a: a + cntall[t], zeros)
    mybase = lax.fori_loop(
        0, wid, lambda t, a: a + cntall[t], zeros)
    padded = ((totals + jnp.full((L,), TB - 1, jnp.int32))
              // jnp.full((L,), TB, jnp.int32)) * jnp.full((L,), TB,
                                                          jnp.int32)
    offs = _csum16(padded) - padded
    base0 = offs + mybase

    ngrp = TPT // L

    def assign(k, basevec):
        def body(g, basevec):
            off = g * L
            idv = ibuf[k, pl.ds(off, L)]
            wv = wbuf[k, pl.ds(off, L)]
            posv = zeros
            for e in range(E):
                mask = idv == evecs[e]
                cums = _csum16(jnp.where(mask, ivec, zeros))
                pc = _bcast16(cums, L - 1)
                bvec = _bcast16(basevec, e)
                posv = posv + jnp.where(mask, bvec + cums - ivec, zeros)
                basevec = basevec + jnp.where(lane == evecs[e], pc, zeros)
            posb[k, pl.ds(off, L)] = posv + jnp.full((L,), T, jnp.int32)
            j = k * TPT + off
            spos2[j // 128, pl.ds(j % 128, L)] = posv
            row = jnp.full((L,), j, jnp.int32) + lane
            tok = jnp.full((L,), t0, jnp.int32) + jnp.full(
                (L,), off, jnp.int32) + lane
            plsc.store_scatter(scomb, [row, zeros], tok)
            plsc.store_scatter(scomb, [row, ivec],
                               plsc.bitcast(wv, jnp.int32))
            return basevec
        return lax.fori_loop(0, ngrp, body, basevec)

    basevec = base0
    for k in range(TOPK):
        basevec = assign(k, basevec)

    NCH = (TOPK * TPT) // 128
    for j in range(NCH):
        pltpu.async_copy(scomb.at[pl.ds(j * 128, 128)],
                         comb.at[spos2.at[j]], sem).wait()
    for k in range(TOPK):
        pltpu.sync_copy(posb.at[k], pos.at[k, pl.ds(t0, TPT)])

    @pl.when(wid == 0)
    def _gmap():
        n_sh = T // TB
        for v in range(64 // L):
            iv = lane + jnp.full((L,), v * L, jnp.int32)
            s_rt = (iv - jnp.full((L,), n_sh, jnp.int32)) * jnp.full(
                (L,), TB, jnp.int32)
            g = jnp.full((L,), SKIP, jnp.int32)
            for e in range(E):
                lo = _bcast16(offs, e)
                hi = lo + _bcast16(padded, e)
                g = jnp.where((s_rt >= lo) & (s_rt < hi), evecs[e], g)
            g = jnp.where(iv < jnp.full((L,), n_sh, jnp.int32),
                          jnp.full((L,), E, jnp.int32), g)
            gmapb[pl.ds(v * L, L)] = g
        pltpu.sync_copy(gmapb, gmap)


def _k2b_dispatch(cnt, idx01, w01, T, E, P_rt):
    TPT = T // NT
    mesh = plsc.VectorSubcoreMesh(core_axis_name="c", subcore_axis_name="s")
    body = functools.partial(_k2b_body, E, TPT, T, P_rt)
    fn = pl.kernel(
        body,
        out_type=[
            jax.ShapeDtypeStruct((P_rt, 128), jnp.int32),
            jax.ShapeDtypeStruct((TOPK, T), jnp.int32),
            jax.ShapeDtypeStruct((64,), jnp.int32),
        ],
        mesh=mesh,
        compiler_params=pltpu.CompilerParams(
            needs_layout_passes=False),
        scratch_types=[
            pltpu.VMEM((NT, L), jnp.int32),
            pltpu.VMEM((TOPK, TPT), jnp.int32),
            pltpu.VMEM((TOPK, TPT), jnp.float32),
            pltpu.VMEM((TOPK, TPT), jnp.int32),
            pltpu.VMEM(((TOPK * TPT) // 128, 128), jnp.int32),
            pltpu.VMEM((TOPK * TPT, 128), jnp.int32),
            pltpu.VMEM((64,), jnp.int32),
            pltpu.SemaphoreType.DMA,
        ],
    )
    return fn(cnt, idx01, w01)


def _k3_body(T, H, P_rt, RPT, comb, xh, xs, combc, idxc, rbuf, sem0, sem1):
    wid = lax.axis_index("s") * NC + lax.axis_index("c")
    r0 = wid * RPT
    CH = 40
    nch = RPT // CH
    zeros = jnp.zeros((L,), jnp.int32)
    tmax = jnp.full((L,), T - 1, jnp.int32)
    sems = [sem0, sem1]
    cps = [None, None]

    def prep(c, p):
        pltpu.sync_copy(comb.at[pl.ds(r0 + c * CH, CH)], combc.at[p])
        for g in range(CH // L + 1):
            gl = min(g * L, CH - L)
            rvec = jnp.full((L,), gl, jnp.int32) + _iota()
            tok = plsc.load_gather(combc.at[p], [rvec, zeros])
            idxc[p, pl.ds(gl, L)] = jnp.minimum(jnp.maximum(tok, zeros),
                                                tmax)
        cps[p] = pltpu.async_copy(xh.at[idxc.at[p]], rbuf.at[p], sems[p])

    prep(0, 0)
    for c in range(1, nch):
        p = c % 2
        prep(c, p)
        cps[1 - p].wait()
        pltpu.sync_copy(rbuf.at[1 - p],
                        xs.at[pl.ds(r0 + (c - 1) * CH, CH)])
    cps[(nch - 1) % 2].wait()
    pltpu.sync_copy(rbuf.at[(nch - 1) % 2],
                    xs.at[pl.ds(r0 + (nch - 1) * CH, CH)])


def _k3_gather(comb, x, T, H, P_rt):
    RPT = P_rt // NT
    mesh = plsc.VectorSubcoreMesh(core_axis_name="c", subcore_axis_name="s")
    body = functools.partial(_k3_body, T, H, P_rt, RPT)
    fn = pl.kernel(
        body,
        out_type=jax.ShapeDtypeStruct((P_rt, H), jnp.float32),
        mesh=mesh,
        compiler_params=pltpu.CompilerParams(
            needs_layout_passes=False),
        scratch_types=[
            pltpu.VMEM((2, 40, 128), jnp.int32),
            pltpu.VMEM((2, 40), jnp.int32),
            pltpu.VMEM((2, 40, H), jnp.float32),
            pltpu.SemaphoreType.DMA,
            pltpu.SemaphoreType.DMA,
        ],
    )
    return fn(comb, x)


def _k4_body(E, n_sh, gm_ref, x_ref, xs_ref, w1_ref, w2_ref, cw_ref,
             ys_ref):
    t = pl.program_id(0)
    ib = pl.program_id(1)
    g = gm_ref[t]

    @pl.when(g != SKIP)
    def _compute():
        xb = jnp.where(t < n_sh, x_ref[...], xs_ref[...])
        w1c = w1_ref[0]
        w2c = w2_ref[0]
        h = lax.dot_general(xb, w1c, (((1,), (1,)), ((), ())),
                            preferred_element_type=jnp.float32)
        h = h * (1.0 / (1.0 + jnp.exp(-h)))
        y = lax.dot_general(h, w2c, (((1,), (1,)), ((), ())),
                            preferred_element_type=jnp.float32)
        c = jnp.where(
            t < n_sh, 1.0,
            lax.bitcast_convert_type(cw_ref[:, 1:2], jnp.float32))
        y = y * c

        @pl.when(ib == 0)
        def _init():
            ys_ref[...] = y

        @pl.when(ib > 0)
        def _acc():
            ys_ref[...] += y


def _k4_ffn(gmap, x, xs, w1, w2, comb, T, H, I, P_rt):
    n_sh = T // TB
    n_tiles = (T + P_rt) // TB
    n_ib = I // IB
    body = functools.partial(_k4_body, w1.shape[0], n_sh)
    grid_spec = pltpu.PrefetchScalarGridSpec(
        num_scalar_prefetch=1,
        grid=(n_tiles, n_ib),
        in_specs=[
            pl.BlockSpec((TB, H),
                         lambda t, ib, gm: (jnp.minimum(t, n_sh - 1), 0)),
            pl.BlockSpec((TB, H),
                         lambda t, ib, gm: (jnp.maximum(t - n_sh, 0), 0)),
            pl.BlockSpec((1, IB, H),
                         lambda t, ib, gm: (jnp.minimum(gm[t], 8), ib, 0)),
            pl.BlockSpec((1, H, IB),
                         lambda t, ib, gm: (jnp.minimum(gm[t], 8), 0, ib)),
            pl.BlockSpec((TB, 128),
                         lambda t, ib, gm: (jnp.maximum(t - n_sh, 0), 0)),
        ],
        out_specs=pl.BlockSpec((TB, H), lambda t, ib, gm: (t, 0)),
    )
    return pl.pallas_call(
        body,
        grid_spec=grid_spec,
        out_shape=jax.ShapeDtypeStruct((T + P_rt, H), jnp.float32),
        compiler_params=pltpu.CompilerParams(
            dimension_semantics=("arbitrary", "arbitrary")),
    )(gmap, x, xs, w1, w2, comb)


def _k5_body(T, H, P, ys, pos, out, praw, p0c, p1c, bbuf, g0, g1, sem0,
             sem1, semb):
    wid = lax.axis_index("s") * NC + lax.axis_index("c")
    TPT = T // NT
    t0 = wid * TPT
    CH = 16
    nch = TPT // CH

    pltpu.sync_copy(pos.at[0, pl.ds(t0, TPT)], praw.at[0])
    pltpu.sync_copy(pos.at[1, pl.ds(t0, TPT)], praw.at[1])

    def ldpos(c, _):
        o = c * CH
        p0 = praw[0, pl.ds(o, L)]
        p1 = praw[1, pl.ds(o, L)]
        p0c[c, ...] = jnp.clip(p0, 0, P - 1)
        p1c[c, ...] = jnp.clip(p1, 0, P - 1)
        return 0

    lax.fori_loop(0, nch, ldpos, 0)

    cps = [None, None]

    def fire(c, p):
        cps[p] = (pltpu.async_copy(ys.at[p0c.at[c]], g0.at[p], sem0),
                  pltpu.async_copy(ys.at[p1c.at[c]], g1.at[p], sem1),
                  pltpu.async_copy(ys.at[pl.ds(t0 + c * CH, CH)],
                                   bbuf.at[p], semb))

    def finish(c, p):
        for cp in cps[p]:
            cp.wait()

        def add(i, _):
            r = i // (H // L)
            o = (i % (H // L)) * L
            bbuf[p, r, pl.ds(o, L)] = (bbuf[p, r, pl.ds(o, L)]
                                       + g0[p, r, pl.ds(o, L)]
                                       + g1[p, r, pl.ds(o, L)])
            return 0

        lax.fori_loop(0, CH * (H // L), add, 0)
        pltpu.sync_copy(bbuf.at[p], out.at[pl.ds(t0 + c * CH, CH)])

    fire(0, 0)
    for c in range(1, nch):
        p = c % 2
        fire(c, p)
        finish(c - 1, 1 - p)
    finish(nch - 1, (nch - 1) % 2)


def _k5_combine(ys, pos, T, H, P):
    TPT = T // NT
    CH = 16
    mesh = plsc.VectorSubcoreMesh(core_axis_name="c", subcore_axis_name="s")
    body = functools.partial(_k5_body, T, H, P)
    fn = pl.kernel(
        body,
        out_type=jax.ShapeDtypeStruct((T, H), jnp.float32),
        mesh=mesh,
        compiler_params=pltpu.CompilerParams(
            needs_layout_passes=False),
        scratch_types=[
            pltpu.VMEM((TOPK, TPT), jnp.int32),
            pltpu.VMEM((TPT // CH, CH), jnp.int32),
            pltpu.VMEM((TPT // CH, CH), jnp.int32),
            pltpu.VMEM((2, CH, H), jnp.float32),
            pltpu.VMEM((2, CH, H), jnp.float32),
            pltpu.VMEM((2, CH, H), jnp.float32),
            pltpu.SemaphoreType.DMA,
            pltpu.SemaphoreType.DMA,
            pltpu.SemaphoreType.DMA,
        ],
    )
    return fn(ys, pos)


def kernel(hidden_states, shared_w1, shared_w2, routed_w1, routed_w2,
           router_w):
    bsz, seq, H = hidden_states.shape
    T = bsz * seq
    E, I, _ = routed_w1.shape
    P_rt = TOPK * T + E * TB

    x = hidden_states.reshape(T, H)
    w1 = jnp.concatenate([routed_w1, shared_w1], axis=0)
    w2 = jnp.concatenate([routed_w2, shared_w2], axis=0)

    logitsT = _k1_router_logits(x, router_w, T, H, E)
    idx01, w01, cnt = _k2a_route(logitsT, T, E)
    comb, pos, gmap = _k2b_dispatch(cnt, idx01, w01, T, E, P_rt)
    xs = _k3_gather(comb, x, T, H, P_rt)
    comb, pos, gmap = _k2b_dispatch(cnt, idx01, w01, T, E, P_rt)
    xs = _k3_gather(comb, x, T, H, P_rt)
    ys = _k4_ffn(gmap, x, xs, w1, w2, comb, T, H, I, P_rt)
    out = _k5_combine(ys, pos, T, H, T + P_rt)
    return out.reshape(bsz, seq, H)


# speedup vs baseline: 1.6512x; 1.0033x over previous
import functools
import jax
import jax.numpy as jnp
from jax import lax
from jax.experimental import pallas as pl
from jax.experimental.pallas import tpu as pltpu
from jax.experimental.pallas import tpu_sc as plsc

NC, NS, L = 2, 16, 16
NT = NC * NS
TOPK = 2
TB = 512
IB = 2048
SKIP = 9


def _f32(x):
    return x.astype(jnp.float32)


def _iota():
    return lax.iota(jnp.int32, L)


def _perm16(v, idx):
    return v.at[idx].get(mode="promise_in_bounds")


def _csum16(v):
    io = _iota()
    zeros = jnp.zeros((L,), jnp.int32)
    r = v
    sh = 1
    while sh < L:
        shv = jnp.full((L,), sh, jnp.int32)
        sv = _perm16(r, jnp.maximum(io - shv, zeros))
        r = r + jnp.where(io >= shv, sv, zeros)
        sh *= 2
    return r


def _bcast16(v, i):
    return _perm16(v, jnp.full((L,), i, jnp.int32))


def _k1_body(rw_ref, x_ref, lt_ref):
    lt_ref[...] = lax.dot_general(
        rw_ref[...], x_ref[...], (((1,), (1,)), ((), ())),
        preferred_element_type=jnp.float32)


def _k1_router_logits(x, router_w, T, H, E):
    TBK = 2048
    return pl.pallas_call(
        _k1_body,
        grid=(T // TBK,),
        in_specs=[
            pl.BlockSpec((E, H), lambda t: (0, 0)),
            pl.BlockSpec((TBK, H), lambda t: (t, 0)),
        ],
        out_specs=pl.BlockSpec((E, TBK), lambda t: (0, t)),
        out_shape=jax.ShapeDtypeStruct((E, T), jnp.float32),
    )(router_w, x)


def _k2a_body(E, TPT, lt_hbm, idx01, w01, cnt, lbuf, i0b, i1b, w0b, w1b,
              cbuf):
    wid = lax.axis_index("s") * NC + lax.axis_index("c")
    t0 = wid * TPT
    for e in range(E):
        pltpu.sync_copy(lt_hbm.at[e, pl.ds(t0, TPT)], lbuf.at[e])

    lane = lax.iota(jnp.int32, L)
    zeros = jnp.zeros((L,), jnp.int32)

    ones = jnp.ones((L,), jnp.float32)
    neg = jnp.full((L,), -3e38, jnp.float32)
    evecs = [jnp.full((L,), e, jnp.int32) for e in range(E)]

    def group(g, accs):
        off = g * L
        ls = [lbuf[e, pl.ds(off, L)] for e in range(E)]
        m0 = ls[0]
        for e in range(1, E):
            m0 = jnp.maximum(m0, ls[e])
        i0 = jnp.full((L,), 99, jnp.int32)
        for e in range(E - 1, -1, -1):
            i0 = jnp.where(ls[e] == m0, evecs[e], i0)
        m1 = neg
        for e in range(E):
            le = jnp.where(i0 == evecs[e], neg, ls[e])
            m1 = jnp.maximum(m1, le)
        i1 = jnp.full((L,), 99, jnp.int32)
        for e in range(E - 1, -1, -1):
            i1 = jnp.where((i0 != evecs[e]) & (ls[e] == m1), evecs[e], i1)
        c0 = ones / (ones + jnp.exp(m1 - m0))
        i0b[pl.ds(off, L)] = i0
        i1b[pl.ds(off, L)] = i1
        w0b[pl.ds(off, L)] = c0
        w1b[pl.ds(off, L)] = ones - c0
        ivec = jnp.ones((L,), jnp.int32)
        zvec = jnp.zeros((L,), jnp.int32)
        new = [accs[e]
               + jnp.where(i0 == evecs[e], ivec, zvec)
               + jnp.where(i1 == evecs[e], ivec, zvec)
               for e in range(E)]
        return new

    accs = [jnp.zeros((L,), jnp.int32) for _ in range(E)]
    accs = lax.fori_loop(0, TPT // L, group, accs)
    lane = _iota()
    zeros = jnp.zeros((L,), jnp.int32)
    cvec = zeros
    for e in range(E):
        tot = _bcast16(_csum16(accs[e]), L - 1)
        cvec = cvec + jnp.where(lane == evecs[e], tot, zeros)
    cbuf[...] = cvec
    pltpu.sync_copy(i0b, idx01.at[0, pl.ds(t0, TPT)])
    pltpu.sync_copy(i1b, idx01.at[1, pl.ds(t0, TPT)])
    pltpu.sync_copy(w0b, w01.at[0, pl.ds(t0, TPT)])
    pltpu.sync_copy(w1b, w01.at[1, pl.ds(t0, TPT)])
    pltpu.sync_copy(cbuf, cnt.at[wid])


def _k2a_route(logitsT, T, E):
    TPT = T // NT
    mesh = plsc.VectorSubcoreMesh(core_axis_name="c", subcore_axis_name="s")
    body = functools.partial(_k2a_body, E, TPT)
    fn = pl.kernel(
        body,
        out_type=[
            jax.ShapeDtypeStruct((TOPK, T), jnp.int32),
            jax.ShapeDtypeStruct((TOPK, T), jnp.float32),
            jax.ShapeDtypeStruct((NT, L), jnp.int32),
        ],
        mesh=mesh,
        compiler_params=pltpu.CompilerParams(
            needs_layout_passes=False),
        scratch_types=[
            pltpu.VMEM((E, TPT), jnp.float32),
            pltpu.VMEM((TPT,), jnp.int32),
            pltpu.VMEM((TPT,), jnp.int32),
            pltpu.VMEM((TPT,), jnp.float32),
            pltpu.VMEM((TPT,), jnp.float32),
            pltpu.VMEM((L,), jnp.int32),
        ],
    )
    return fn(logitsT)


def _k2b_body(E, TPT, T, P_rt, cnt, idx01h, w01h, comb, pos, gmap,
              cntall, ibuf, wbuf, posb, spos2, scomb, gmapb, sem):
    wid = lax.axis_index("s") * NC + lax.axis_index("c")
    t0 = wid * TPT
    pltpu.sync_copy(cnt, cntall)
    for k in range(TOPK):
        pltpu.sync_copy(idx01h.at[k, pl.ds(t0, TPT)], ibuf.at[k])
        pltpu.sync_copy(w01h.at[k, pl.ds(t0, TPT)], wbuf.at[k])

    lane = _iota()
    zeros = jnp.zeros((L,), jnp.int32)
    ivec = jnp.ones((L,), jnp.int32)
    evecs = [jnp.full((L,), e, jnp.int32) for e in range(E)]

    totals = lax.fori_loop(
        0, NT, lambda t, a: a + cntall[t], zeros)
    mybase = lax.fori_loop(
        0, wid, lambda t, a: a + cntall[t], zeros)
    padded = ((totals + jnp.full((L,), TB - 1, jnp.int32))
              // jnp.full((L,), TB, jnp.int32)) * jnp.full((L,), TB,
                                                          jnp.int32)
    offs = _csum16(padded) - padded
    base0 = offs + mybase

    ngrp = TPT // L

    def assign(k, basevec):
        def body(g, basevec):
            off = g * L
            idv = ibuf[k, pl.ds(off, L)]
            wv = wbuf[k, pl.ds(off, L)]
            posv = zeros
            for e in range(E):
                mask = idv == evecs[e]
                cums = _csum16(jnp.where(mask, ivec, zeros))
                pc = _bcast16(cums, L - 1)
                bvec = _bcast16(basevec, e)
                posv = posv + jnp.where(mask, bvec + cums - ivec, zeros)
                basevec = basevec + jnp.where(lane == evecs[e], pc, zeros)
            posb[k, pl.ds(off, L)] = posv + jnp.full((L,), T, jnp.int32)
            j = k * TPT + off
            spos2[j // 128, pl.ds(j % 128, L)] = posv
            row = jnp.full((L,), j, jnp.int32) + lane
            tok = jnp.full((L,), t0, jnp.int32) + jnp.full(
                (L,), off, jnp.int32) + lane
            plsc.store_scatter(scomb, [row, zeros], tok)
            plsc.store_scatter(scomb, [row, ivec],
                               plsc.bitcast(wv, jnp.int32))
            return basevec
        return lax.fori_loop(0, ngrp, body, basevec)

    basevec = base0
    for k in range(TOPK):
        basevec = assign(k, basevec)

    NCH = (TOPK * TPT) // 128
    for j in range(NCH):
        pltpu.async_copy(scomb.at[pl.ds(j * 128, 128)],
                         comb.at[spos2.at[j]], sem).wait()
    for k in range(TOPK):
        pltpu.sync_copy(posb.at[k], pos.at[k, pl.ds(t0, TPT)])

    @pl.when(wid == 0)
    def _gmap():
        n_sh = T // TB
        for v in range(64 // L):
            iv = lane + jnp.full((L,), v * L, jnp.int32)
            s_rt = (iv - jnp.full((L,), n_sh, jnp.int32)) * jnp.full(
                (L,), TB, jnp.int32)
            g = jnp.full((L,), SKIP, jnp.int32)
            for e in range(E):
                lo = _bcast16(offs, e)
                hi = lo + _bcast16(padded, e)
                g = jnp.where((s_rt >= lo) & (s_rt < hi), evecs[e], g)
            g = jnp.where(iv < jnp.full((L,), n_sh, jnp.int32),
                          jnp.full((L,), E, jnp.int32), g)
            gmapb[pl.ds(v * L, L)] = g
        pltpu.sync_copy(gmapb, gmap)


def _k2b_dispatch(cnt, idx01, w01, T, E, P_rt):
    TPT = T // NT
    mesh = plsc.VectorSubcoreMesh(core_axis_name="c", subcore_axis_name="s")
    body = functools.partial(_k2b_body, E, TPT, T, P_rt)
    fn = pl.kernel(
        body,
        out_type=[
            jax.ShapeDtypeStruct((P_rt, 128), jnp.int32),
            jax.ShapeDtypeStruct((TOPK, T), jnp.int32),
            jax.ShapeDtypeStruct((64,), jnp.int32),
        ],
        mesh=mesh,
        compiler_params=pltpu.CompilerParams(
            needs_layout_passes=False),
        scratch_types=[
            pltpu.VMEM((NT, L), jnp.int32),
            pltpu.VMEM((TOPK, TPT), jnp.int32),
            pltpu.VMEM((TOPK, TPT), jnp.float32),
            pltpu.VMEM((TOPK, TPT), jnp.int32),
            pltpu.VMEM(((TOPK * TPT) // 128, 128), jnp.int32),
            pltpu.VMEM((TOPK * TPT, 128), jnp.int32),
            pltpu.VMEM((64,), jnp.int32),
            pltpu.SemaphoreType.DMA,
        ],
    )
    return fn(cnt, idx01, w01)


def _k3_body(T, H, P_rt, RPT, comb, xh, xs, combc, idxc, rbuf, sem0, sem1,
             wsem0, wsem1):
    wid = lax.axis_index("s") * NC + lax.axis_index("c")
    r0 = wid * RPT
    CH = 40
    nch = RPT // CH
    zeros = jnp.zeros((L,), jnp.int32)
    tmax = jnp.full((L,), T - 1, jnp.int32)
    sems = [sem0, sem1]
    wsems = [wsem0, wsem1]
    cps = [None, None]

    def prep(c, p):
        pltpu.sync_copy(comb.at[pl.ds(r0 + c * CH, CH)], combc.at[p])
        for g in range(CH // L + 1):
            gl = min(g * L, CH - L)
            rvec = jnp.full((L,), gl, jnp.int32) + _iota()
            tok = plsc.load_gather(combc.at[p], [rvec, zeros])
            idxc[p, pl.ds(gl, L)] = jnp.minimum(jnp.maximum(tok, zeros),
                                                tmax)
        cps[p] = pltpu.async_copy(xh.at[idxc.at[p]], rbuf.at[p], sems[p])

    wcps = [None, None]
    for c in range(nch):
        p = c % 2
        if c >= 2:
            wcps[p].wait()
        prep(c, p)
        if c >= 1:
            cps[1 - p].wait()
            wcps[1 - p] = pltpu.async_copy(
                rbuf.at[1 - p], xs.at[pl.ds(r0 + (c - 1) * CH, CH)],
                wsems[1 - p])
    pl_ = (nch - 1) % 2
    cps[pl_].wait()
    wcps[pl_] = pltpu.async_copy(
        rbuf.at[pl_], xs.at[pl.ds(r0 + (nch - 1) * CH, CH)], wsems[pl_])
    wcps[0].wait()
    wcps[1].wait()


def _k3_gather(comb, x, T, H, P_rt):
    RPT = P_rt // NT
    mesh = plsc.VectorSubcoreMesh(core_axis_name="c", subcore_axis_name="s")
    body = functools.partial(_k3_body, T, H, P_rt, RPT)
    fn = pl.kernel(
        body,
        out_type=jax.ShapeDtypeStruct((P_rt, H), jnp.float32),
        mesh=mesh,
        compiler_params=pltpu.CompilerParams(
            needs_layout_passes=False),
        scratch_types=[
            pltpu.VMEM((2, 40, 128), jnp.int32),
            pltpu.VMEM((2, 40), jnp.int32),
            pltpu.VMEM((2, 40, H), jnp.float32),
            pltpu.SemaphoreType.DMA,
            pltpu.SemaphoreType.DMA,
            pltpu.SemaphoreType.DMA,
            pltpu.SemaphoreType.DMA,
        ],
    )
    return fn(comb, x)


def _k4_body(E, n_sh, gm_ref, x_ref, xs_ref, w1_ref, w2_ref, cw_ref,
             ys_ref):
    t = pl.program_id(0)
    ib = pl.program_id(1)
    g = gm_ref[t]

    @pl.when(g != SKIP)
    def _compute():
        xb = jnp.where(t < n_sh, x_ref[...], xs_ref[...])
        w1c = w1_ref[0]
        w2c = w2_ref[0]
        h = lax.dot_general(xb, w1c, (((1,), (1,)), ((), ())),
                            preferred_element_type=jnp.float32)
        h = h * (1.0 / (1.0 + jnp.exp(-h)))
        y = lax.dot_general(h, w2c, (((1,), (1,)), ((), ())),
                            preferred_element_type=jnp.float32)
        c = jnp.where(
            t < n_sh, 1.0,
            lax.bitcast_convert_type(cw_ref[:, 1:2], jnp.float32))
        y = y * c

        @pl.when(ib == 0)
        def _init():
            ys_ref[...] = y

        @pl.when(ib > 0)
        def _acc():
            ys_ref[...] += y


def _k4_ffn(gmap, x, xs, w1, w2, comb, T, H, I, P_rt):
    n_sh = T // TB
    n_tiles = (T + P_rt) // TB
    n_ib = I // IB
    body = functools.partial(_k4_body, w1.shape[0], n_sh)
    grid_spec = pltpu.PrefetchScalarGridSpec(
        num_scalar_prefetch=1,
        grid=(n_tiles, n_ib),
        in_specs=[
            pl.BlockSpec((TB, H),
                         lambda t, ib, gm: (jnp.minimum(t, n_sh - 1), 0)),
            pl.BlockSpec((TB, H),
                         lambda t, ib, gm: (jnp.maximum(t - n_sh, 0), 0)),
            pl.BlockSpec((1, IB, H),
                         lambda t, ib, gm: (jnp.minimum(gm[t], 8), ib, 0)),
            pl.BlockSpec((1, H, IB),
                         lambda t, ib, gm: (jnp.minimum(gm[t], 8), 0, ib)),
            pl.BlockSpec((TB, 128),
                         lambda t, ib, gm: (jnp.maximum(t - n_sh, 0), 0)),
        ],
        out_specs=pl.BlockSpec((TB, H), lambda t, ib, gm: (t, 0)),
    )
    return pl.pallas_call(
        body,
        grid_spec=grid_spec,
        out_shape=jax.ShapeDtypeStruct((T + P_rt, H), jnp.float32),
        compiler_params=pltpu.CompilerParams(
            dimension_semantics=("arbitrary", "arbitrary")),
    )(gmap, x, xs, w1, w2, comb)


def _k5_body(T, H, P, ys, pos, out, praw, p0c, p1c, bbuf, g0, g1, sem0,
             sem1, semb, semo0, semo1):
    wid = lax.axis_index("s") * NC + lax.axis_index("c")
    TPT = T // NT
    t0 = wid * TPT
    CH = 16
    nch = TPT // CH

    pltpu.sync_copy(pos.at[0, pl.ds(t0, TPT)], praw.at[0])
    pltpu.sync_copy(pos.at[1, pl.ds(t0, TPT)], praw.at[1])

    def ldpos(c, _):
        o = c * CH
        p0 = praw[0, pl.ds(o, L)]
        p1 = praw[1, pl.ds(o, L)]
        p0c[c, ...] = jnp.clip(p0, 0, P - 1)
        p1c[c, ...] = jnp.clip(p1, 0, P - 1)
        return 0

    lax.fori_loop(0, nch, ldpos, 0)

    cps = [None, None]
    wcps = [None, None]
    semo = [semo0, semo1]

    def fire(c, p):
        cps[p] = (pltpu.async_copy(ys.at[p0c.at[c]], g0.at[p], sem0),
                  pltpu.async_copy(ys.at[p1c.at[c]], g1.at[p], sem1),
                  pltpu.async_copy(ys.at[pl.ds(t0 + c * CH, CH)],
                                   bbuf.at[p], semb))

    def finish(c, p):
        for cp in cps[p]:
            cp.wait()

        def add(i, _):
            r = i // (H // L)
            o = (i % (H // L)) * L
            bbuf[p, r, pl.ds(o, L)] = (bbuf[p, r, pl.ds(o, L)]
                                       + g0[p, r, pl.ds(o, L)]
                                       + g1[p, r, pl.ds(o, L)])
            return 0

        lax.fori_loop(0, CH * (H // L), add, 0)
        wcps[p] = pltpu.async_copy(bbuf.at[p],
                                   out.at[pl.ds(t0 + c * CH, CH)], semo[p])

    for c in range(nch):
        p = c % 2
        if c >= 2:
            wcps[p].wait()
        fire(c, p)
        if c >= 1:
            finish(c - 1, 1 - p)
    finish(nch - 1, (nch - 1) % 2)
    wcps[0].wait()
    wcps[1].wait()


def _k5_combine(ys, pos, T, H, P):
    TPT = T // NT
    CH = 16
    mesh = plsc.VectorSubcoreMesh(core_axis_name="c", subcore_axis_name="s")
    body = functools.partial(_k5_body, T, H, P)
    fn = pl.kernel(
        body,
        out_type=jax.ShapeDtypeStruct((T, H), jnp.float32),
        mesh=mesh,
        compiler_params=pltpu.CompilerParams(
            needs_layout_passes=False),
        scratch_types=[
            pltpu.VMEM((TOPK, TPT), jnp.int32),
            pltpu.VMEM((TPT // CH, CH), jnp.int32),
            pltpu.VMEM((TPT // CH, CH), jnp.int32),
            pltpu.VMEM((2, CH, H), jnp.float32),
            pltpu.VMEM((2, CH, H), jnp.float32),
            pltpu.VMEM((2, CH, H), jnp.float32),
            pltpu.SemaphoreType.DMA,
            pltpu.SemaphoreType.DMA,
            pltpu.SemaphoreType.DMA,
            pltpu.SemaphoreType.DMA,
            pltpu.SemaphoreType.DMA,
        ],
    )
    return fn(ys, pos)


def kernel(hidden_states, shared_w1, shared_w2, routed_w1, routed_w2,
           router_w):
    bsz, seq, H = hidden_states.shape
    T = bsz * seq
    E, I, _ = routed_w1.shape
    P_rt = TOPK * T + E * TB

    x = hidden_states.reshape(T, H)
    w1 = jnp.concatenate([routed_w1, shared_w1], axis=0)
    w2 = jnp.concatenate([routed_w2, shared_w2], axis=0)

    logitsT = _k1_router_logits(x, router_w, T, H, E)
    idx01, w01, cnt = _k2a_route(logitsT, T, E)
    comb, pos, gmap = _k2b_dispatch(cnt, idx01, w01, T, E, P_rt)
    xs = _k3_gather(comb, x, T, H, P_rt)
    comb, pos, gmap = _k2b_dispatch(cnt, idx01, w01, T, E, P_rt)
    xs = _k3_gather(comb, x, T, H, P_rt)
    ys = _k4_ffn(gmap, x, xs, w1, w2, comb, T, H, I, P_rt)
    out = _k5_combine(ys, pos, T, H, T + P_rt)
    return out.reshape(bsz, seq, H)
